# baseline (speedup 1.0000x reference)
import jax
import jax.numpy as jnp
from jax.experimental import pallas as pl
from jax.experimental.pallas import tpu as pltpu

_B = 128
_F = 256
_C0 = 256
_C1 = 64
_N1 = 16
_TB = 32
_STEPS = _B // _TB
_EPS = 1e-5
_F32 = jnp.float32
_BF = jnp.bfloat16


def _chunk_mask(dd, k):
    l = dd // k
    j = jax.lax.broadcasted_iota(jnp.int32, (dd, k), 0)
    kk = jax.lax.broadcasted_iota(jnp.int32, (dd, k), 1)
    return (j // l == kk).astype(_F32)


def _acc_stats(t, k, s_ref, q_ref):
    dd = t.shape[-1]
    s = jnp.sum(t, axis=0)
    q = jnp.sum(t * t, axis=0)
    if k == dd:
        s_ref[...] += s
        q_ref[...] += q
    else:
        m = _chunk_mask(dd, k)
        s_ref[...] += jnp.dot(s, m, preferred_element_type=_F32)
        q_ref[...] += jnp.dot(q, m, preferred_element_type=_F32)


def _expand(s_ref, q_ref, n, a_ref, sh_ref=None, cnt_ref=None, rows=None,
            cm_ref=None):
    s = s_ref[...] if rows is None else s_ref[0:rows, :]
    q = q_ref[...] if rows is None else q_ref[0:rows, :]
    inv = 1.0 / n
    mean = s * inv
    var = q * inv - mean * mean
    rstd = jax.lax.rsqrt(var + _EPS)
    if cnt_ref is not None:
        rstd = rstd / cnt_ref[...]
    sh = mean * rstd
    l, k = mean.shape
    dd = a_ref.shape[-1]
    w = dd // k
    if k == dd:
        a_ref[...] = rstd
        if sh_ref is not None:
            sh_ref[...] = sh
        if cm_ref is not None:
            cm_ref[...] = sh
        return
    for c in range(k):
        a_ref[:, c * w:(c + 1) * w] = jnp.broadcast_to(rstd[:, c:c + 1], (l, w))
        shc = jnp.broadcast_to(sh[:, c:c + 1], (l, w))
        if sh_ref is not None:
            sh_ref[:, c * w:(c + 1) * w] = shc
        if cm_ref is not None:
            cm_ref[:, c * w:(c + 1) * w] = jnp.mean(shc, axis=0, keepdims=True)


def _stage1(n0, n1, n2, n3, wp, w30, w20, w10, w00, bp, b30, b20, b10, b00,
            t1_o, ty_o, tz_o, tw_o, s3, q3, s2, q2, s1, q1, s0, q0,
            wc3, wc2, wc1, wc0, bc3, bc2, bc1, bc0):
    i = pl.program_id(0)

    @pl.when(i == 0)
    def _():
        for r in (s3, q3, s2, q2, s1, q1, s0, q0):
            r[...] = jnp.zeros_like(r)
        p = wp[...]
        bpv = bp[...]
        for wh, bh, wc, bc in ((w30, b30, wc3, bc3), (w20, b20, wc2, bc2),
                               (w10, b10, wc1, bc1), (w00, b00, wc0, bc0)):
            wc[...] = jnp.dot(p, wh[...],
                              preferred_element_type=_F32).astype(_BF)
            bc[...] = jnp.dot(bpv, wh[...], preferred_element_type=_F32) + bh[...]

    def hop(n_ref, wc, bc, o_ref, l):
        x = n_ref[...].reshape(_TB * l, _F).astype(_BF)
        t = jnp.dot(x, wc[...], preferred_element_type=_F32) + bc[...]
        t = t.reshape(_TB, l, t.shape[-1])
        if o_ref is not None:
            o_ref[...] = t.astype(o_ref.dtype)
        return t

    _acc_stats(hop(n0, wc3, bc3, t1_o, _C0), 1, s3, q3)
    _acc_stats(hop(n1, wc2, bc2, ty_o, _C1), 4, s2, q2)
    _acc_stats(hop(n2, wc1, bc1, tz_o, _N1), 16, s1, q1)
    _acc_stats(hop(n3, wc0, bc0, tw_o, 1), 256, s0, q0)


def _stage2(t1_ref, s3, q3, cnt0, w_ref, b_ref, out_ref, s_ref, q_ref,
            a1, sh1, wb):
    i = pl.program_id(0)

    @pl.when(i == 0)
    def _():
        s_ref[...] = jnp.zeros_like(s_ref)
        q_ref[...] = jnp.zeros_like(q_ref)
        _expand(s3, q3, _B * _C0, a1, sh_ref=sh1, cnt_ref=cnt0)
        wb[...] = w_ref[...].astype(_BF)

    x = t1_ref[...].astype(_F32)
    p = jnp.maximum(x * a1[...][None] - sh1[...][None], 0.0)
    t = jnp.dot(p.reshape(_TB * _C0, 256).astype(_BF), wb[...],
                preferred_element_type=_F32) + b_ref[...]
    t = t.reshape(_TB, _C0, 512)
    out_ref[...] = t[:, :_C1, :].astype(out_ref.dtype)
    _acc_stats(t, 2, s_ref, q_ref)


def _stage3(t3_ref, s31, q31, cnt1, w32, b32r, ty1_ref, s2, q2, w21, b21r,
            t5_o, s5, q5, ty2_o, sy, qy, a3, sh3, a2, sh2, w32b, w21b):
    i = pl.program_id(0)

    @pl.when(i == 0)
    def _():
        for r in (s5, q5, sy, qy):
            r[...] = jnp.zeros_like(r)
        _expand(s31, q31, _B * _C0, a3, sh_ref=sh3, cnt_ref=cnt1, rows=_C1)
        _expand(s2, q2, _B * _C1, a2, sh_ref=sh2, cnt_ref=cnt1)
        w32b[...] = w32[...].astype(_BF)
        w21b[...] = w21[...].astype(_BF)

    p = jnp.maximum(t3_ref[...].astype(_F32) * a3[...][None] - sh3[...][None],
                    0.0)
    t5 = jnp.dot(p.reshape(_TB * _C1, 512).astype(_BF), w32b[...],
                 preferred_element_type=_F32) + b32r[...]
    t5 = t5.reshape(_TB, _C1, 1024)
    t5_o[...] = t5.astype(t5_o.dtype)
    _acc_stats(t5, 16, s5, q5)

    p2 = jnp.maximum(ty1_ref[...].astype(_F32) * a2[...][None] - sh2[...][None],
                     0.0)
    ty2 = jnp.dot(p2.reshape(_TB * _C1, 256).astype(_BF), w21b[...],
                  preferred_element_type=_F32) + b21r[...]
    ty2 = ty2.reshape(_TB, _C1, 512)
    ty2_o[...] = ty2.astype(ty2_o.dtype)
    _acc_stats(ty2, 8, sy, qy)


def _stage4(t5_ref, s5, q5, ty2_ref, sy, qy, tz_ref, sz, qz, tw_ref, s0, q0,
            w_ref, b_ref, out_ref, a5, c5, ay, cy, az, cz, a0, c0, wb):
    i = pl.program_id(0)

    @pl.when(i == 0)
    def _():
        _expand(s5, q5, _B * _C1, a5, cm_ref=c5)
        _expand(sy, qy, _B * _C1, ay, cm_ref=cy)
        _expand(sz, qz, _B * _N1, az, cm_ref=cz)
        _expand(s0, q0, _B, a0, cm_ref=c0)
        wb[...] = w_ref[...].astype(_BF)

    h3 = jnp.mean(t5_ref[...].astype(_F32) * a5[...][None], axis=1) - c5[...]
    h2 = jnp.mean(ty2_ref[...].astype(_F32) * ay[...][None], axis=1) - cy[...]
    h1 = jnp.mean(tz_ref[...] * az[...][None], axis=1) - cz[...]
    h0 = jnp.mean(tw_ref[...] * a0[...][None], axis=1) - c0[...]
    h = jnp.concatenate([h0, h1, h2, h3], axis=1)
    out_ref[...] = jnp.dot(h.astype(_BF), wb[...],
                           preferred_element_type=_F32) + b_ref[...]


def _bspec(l, d):
    return pl.BlockSpec((_TB, l, d), lambda i: (i, 0, 0))


def _cspec(r, c):
    return pl.BlockSpec((r, c), lambda i: (0, 0))


def _vmem(shape):
    return pltpu.VMEM(shape, _F32)


def _vmemb(shape):
    return pltpu.VMEM(shape, _BF)


_CP = pltpu.CompilerParams(dimension_semantics=("arbitrary",))


def kernel(nodes_0, nodes_1, nodes_2, nodes_3, counts_0, counts_1,
           W_proj, b_proj, W30, b30, W31, b31, W32, b32,
           W20, b20, W21, b21, W10, b10, W00, b00, W, b):
    f = _F32
    cnt0 = counts_0[0].astype(f)[:, None]
    cnt1 = counts_1[0].astype(f)[:, None]

    s1_out = pl.pallas_call(
        _stage1,
        grid=(_STEPS,),
        in_specs=[
            _bspec(_C0, _F), _bspec(_C1, _F), _bspec(_N1, _F), _bspec(1, _F),
            _cspec(_F, 128),
            _cspec(128, 256), _cspec(128, 256), _cspec(128, 256), _cspec(128, 256),
            _cspec(1, 128),
            _cspec(1, 256), _cspec(1, 256), _cspec(1, 256), _cspec(1, 256),
        ],
        out_specs=[
            _bspec(_C0, 256), _bspec(_C1, 256), _bspec(_N1, 256), _bspec(1, 256),
            _cspec(_C0, 1), _cspec(_C0, 1),
            _cspec(_C1, 4), _cspec(_C1, 4),
            _cspec(_N1, 16), _cspec(_N1, 16),
            _cspec(1, 256), _cspec(1, 256),
        ],
        out_shape=[
            jax.ShapeDtypeStruct((_B, _C0, 256), _BF),
            jax.ShapeDtypeStruct((_B, _C1, 256), _BF),
            jax.ShapeDtypeStruct((_B, _N1, 256), f),
            jax.ShapeDtypeStruct((_B, 1, 256), f),
            jax.ShapeDtypeStruct((_C0, 1), f), jax.ShapeDtypeStruct((_C0, 1), f),
            jax.ShapeDtypeStruct((_C1, 4), f), jax.ShapeDtypeStruct((_C1, 4), f),
            jax.ShapeDtypeStruct((_N1, 16), f), jax.ShapeDtypeStruct((_N1, 16), f),
            jax.ShapeDtypeStruct((1, 256), f), jax.ShapeDtypeStruct((1, 256), f),
        ],
        scratch_shapes=[_vmemb((_F, 256)), _vmemb((_F, 256)),
                        _vmemb((_F, 256)), _vmemb((_F, 256)),
                        _vmem((1, 256)), _vmem((1, 256)),
                        _vmem((1, 256)), _vmem((1, 256))],
        compiler_params=_CP,
    )(nodes_0, nodes_1, nodes_2, nodes_3, W_proj, W30, W20, W10, W00,
      b_proj[None, :], b30[None, :], b20[None, :], b10[None, :], b00[None, :])
    t1, ty1, tz1, tw1, s3, q3, s2, q2, sz, qz, s0, q0 = s1_out

    t3r, s31, q31 = pl.pallas_call(
        _stage2,
        grid=(_STEPS,),
        in_specs=[_bspec(_C0, 256), _cspec(_C0, 1), _cspec(_C0, 1),
                  _cspec(_C0, 1), _cspec(256, 512), _cspec(1, 512)],
        out_specs=[_bspec(_C1, 512), _cspec(_C0, 2), _cspec(_C0, 2)],
        out_shape=[
            jax.ShapeDtypeStruct((_B, _C1, 512), _BF),
            jax.ShapeDtypeStruct((_C0, 2), f), jax.ShapeDtypeStruct((_C0, 2), f),
        ],
        scratch_shapes=[_vmem((_C0, 256)), _vmem((_C0, 256)),
                        _vmemb((256, 512))],
        compiler_params=_CP,
    )(t1, s3, q3, cnt0, W31, b31[None, :])

    t5, s5, q5, ty2, sy, qy = pl.pallas_call(
        _stage3,
        grid=(_STEPS,),
        in_specs=[_bspec(_C1, 512), _cspec(_C0, 2), _cspec(_C0, 2),
                  _cspec(_C1, 1), _cspec(512, 1024), _cspec(1, 1024),
                  _bspec(_C1, 256), _cspec(_C1, 4), _cspec(_C1, 4),
                  _cspec(256, 512), _cspec(1, 512)],
        out_specs=[_bspec(_C1, 1024), _cspec(_C1, 16), _cspec(_C1, 16),
                   _bspec(_C1, 512), _cspec(_C1, 8), _cspec(_C1, 8)],
        out_shape=[
            jax.ShapeDtypeStruct((_B, _C1, 1024), _BF),
            jax.ShapeDtypeStruct((_C1, 16), f), jax.ShapeDtypeStruct((_C1, 16), f),
            jax.ShapeDtypeStruct((_B, _C1, 512), _BF),
            jax.ShapeDtypeStruct((_C1, 8), f), jax.ShapeDtypeStruct((_C1, 8), f),
        ],
        scratch_shapes=[_vmem((_C1, 512)), _vmem((_C1, 512)),
                        _vmem((_C1, 256)), _vmem((_C1, 256)),
                        _vmemb((512, 1024)), _vmemb((256, 512))],
        compiler_params=_CP,
    )(t3r, s31, q31, cnt1, W32, b32[None, :], ty1, s2, q2, W21, b21[None, :])

    out2d = pl.pallas_call(
        _stage4,
        grid=(_STEPS,),
        in_specs=[_bspec(_C1, 1024), _cspec(_C1, 16), _cspec(_C1, 16),
                  _bspec(_C1, 512), _cspec(_C1, 8), _cspec(_C1, 8),
                  _bspec(_N1, 256), _cspec(_N1, 16), _cspec(_N1, 16),
                  _bspec(1, 256), _cspec(1, 256), _cspec(1, 256),
                  _cspec(2048, 2048), _cspec(1, 2048)],
        out_specs=pl.BlockSpec((_TB, 2048), lambda i: (i, 0)),
        out_shape=jax.ShapeDtypeStruct((_B, 2048), f),
        scratch_shapes=[_vmem((_C1, 1024)), _vmem((1, 1024)),
                        _vmem((_C1, 512)), _vmem((1, 512)),
                        _vmem((_N1, 256)), _vmem((1, 256)),
                        _vmem((1, 256)), _vmem((1, 256)),
                        _vmemb((2048, 2048))],
        compiler_params=_CP,
    )(t5, s5, q5, ty2, sy, qy, tz1, sz, qz, tw1, s0, q0, W, b[None, :])

    return out2d.reshape(_B, 2048, 1)


# speedup vs baseline: 1.0320x; 1.0320x over previous
import jax
import jax.numpy as jnp
from jax.experimental import pallas as pl
from jax.experimental.pallas import tpu as pltpu

_B = 128
_F = 256
_C0 = 256
_C1 = 64
_N1 = 16
_TB = 32
_STEPS = _B // _TB
_EPS = 1e-5
_F32 = jnp.float32
_BF = jnp.bfloat16


def _chunk_mask(dd, k):
    l = dd // k
    j = jax.lax.broadcasted_iota(jnp.int32, (dd, k), 0)
    kk = jax.lax.broadcasted_iota(jnp.int32, (dd, k), 1)
    return (j // l == kk).astype(_F32)


def _acc_stats(t, k, s_ref, q_ref):
    dd = t.shape[-1]
    s = jnp.sum(t, axis=0)
    q = jnp.sum(t * t, axis=0)
    if k == dd:
        s_ref[...] += s
        q_ref[...] += q
    else:
        m = _chunk_mask(dd, k)
        s_ref[...] += jnp.dot(s, m, preferred_element_type=_F32)
        q_ref[...] += jnp.dot(q, m, preferred_element_type=_F32)


def _expand(s_ref, q_ref, n, a_ref, sh_ref=None, cnt_ref=None, rows=None,
            cm_ref=None):
    s = s_ref[...] if rows is None else s_ref[0:rows, :]
    q = q_ref[...] if rows is None else q_ref[0:rows, :]
    inv = 1.0 / n
    mean = s * inv
    var = q * inv - mean * mean
    rstd = jax.lax.rsqrt(var + _EPS)
    if cnt_ref is not None:
        rstd = rstd / cnt_ref[...]
    sh = mean * rstd
    l, k = mean.shape
    dd = a_ref.shape[-1]
    w = dd // k
    if k == dd:
        a_ref[...] = rstd
        if sh_ref is not None:
            sh_ref[...] = sh
        if cm_ref is not None:
            cm_ref[...] = sh
        return
    for c in range(k):
        a_ref[:, c * w:(c + 1) * w] = jnp.broadcast_to(rstd[:, c:c + 1], (l, w))
        shc = jnp.broadcast_to(sh[:, c:c + 1], (l, w))
        if sh_ref is not None:
            sh_ref[:, c * w:(c + 1) * w] = shc
        if cm_ref is not None:
            cm_ref[:, c * w:(c + 1) * w] = jnp.mean(shc, axis=0, keepdims=True)


def _stage1(n0, n1, n2, n3, wp, w30, w20, w10, w00, bp, b30, b20, b10, b00,
            t1_o, ty_o, tz_o, tw_o, s3, q3, s2, q2, s1, q1, s0, q0,
            wc3, wc2, wc1, wc0, bc3, bc2, bc1, bc0):
    i = pl.program_id(0)

    @pl.when(i == 0)
    def _():
        for r in (s3, q3, s2, q2, s1, q1, s0, q0):
            r[...] = jnp.zeros_like(r)
        p = wp[...]
        bpv = bp[...]
        for wh, bh, wc, bc in ((w30, b30, wc3, bc3), (w20, b20, wc2, bc2),
                               (w10, b10, wc1, bc1), (w00, b00, wc0, bc0)):
            wc[...] = jnp.dot(p, wh[...], preferred_element_type=_F32)
            bc[...] = jnp.dot(bpv, wh[...], preferred_element_type=_F32) + bh[...]

    def hop(n_ref, wc, bc, o_ref, l):
        x = n_ref[...].reshape(_TB * l, _F)
        t = jnp.dot(x, wc[...], preferred_element_type=_F32) + bc[...]
        t = t.reshape(_TB, l, t.shape[-1])
        if o_ref is not None:
            o_ref[...] = t.astype(o_ref.dtype)
        return t

    _acc_stats(hop(n0, wc3, bc3, t1_o, _C0), 1, s3, q3)
    _acc_stats(hop(n1, wc2, bc2, ty_o, _C1), 4, s2, q2)
    _acc_stats(hop(n2, wc1, bc1, tz_o, _N1), 16, s1, q1)
    _acc_stats(hop(n3, wc0, bc0, tw_o, 1), 256, s0, q0)


def _stage2(t1_ref, s3, q3, cnt0, w_ref, b_ref, out_ref, s_ref, q_ref,
            a1, sh1):
    i = pl.program_id(0)

    @pl.when(i == 0)
    def _():
        s_ref[...] = jnp.zeros_like(s_ref)
        q_ref[...] = jnp.zeros_like(q_ref)
        _expand(s3, q3, _B * _C0, a1, sh_ref=sh1, cnt_ref=cnt0)

    x = t1_ref[...].astype(_F32)
    p = jnp.maximum(x * a1[...][None] - sh1[...][None], 0.0)
    t = jnp.dot(p.reshape(_TB * _C0, 256), w_ref[...],
                preferred_element_type=_F32) + b_ref[...]
    t = t.reshape(_TB, _C0, 512)
    out_ref[...] = t[:, :_C1, :].astype(out_ref.dtype)
    _acc_stats(t, 2, s_ref, q_ref)


def _stage3(t3_ref, s31, q31, cnt1, w32, b32r, ty1_ref, s2, q2, w21, b21r,
            t5_o, s5, q5, ty2_o, sy, qy, a3, sh3, a2, sh2):
    i = pl.program_id(0)

    @pl.when(i == 0)
    def _():
        for r in (s5, q5, sy, qy):
            r[...] = jnp.zeros_like(r)
        _expand(s31, q31, _B * _C0, a3, sh_ref=sh3, cnt_ref=cnt1, rows=_C1)
        _expand(s2, q2, _B * _C1, a2, sh_ref=sh2, cnt_ref=cnt1)

    p = jnp.maximum(t3_ref[...].astype(_F32) * a3[...][None] - sh3[...][None],
                    0.0)
    t5 = jnp.dot(p.reshape(_TB * _C1, 512), w32[...],
                 preferred_element_type=_F32) + b32r[...]
    t5 = t5.reshape(_TB, _C1, 1024)
    t5_o[...] = t5.astype(t5_o.dtype)
    _acc_stats(t5, 16, s5, q5)

    p2 = jnp.maximum(ty1_ref[...].astype(_F32) * a2[...][None] - sh2[...][None],
                     0.0)
    ty2 = jnp.dot(p2.reshape(_TB * _C1, 256), w21[...],
                  preferred_element_type=_F32) + b21r[...]
    ty2 = ty2.reshape(_TB, _C1, 512)
    ty2_o[...] = ty2.astype(ty2_o.dtype)
    _acc_stats(ty2, 8, sy, qy)


def _stage4(t5_ref, s5, q5, ty2_ref, sy, qy, tz_ref, sz, qz, tw_ref, s0, q0,
            w_ref, b_ref, out_ref, a5, c5, ay, cy, az, cz, a0, c0):
    i = pl.program_id(0)

    @pl.when(i == 0)
    def _():
        _expand(s5, q5, _B * _C1, a5, cm_ref=c5)
        _expand(sy, qy, _B * _C1, ay, cm_ref=cy)
        _expand(sz, qz, _B * _N1, az, cm_ref=cz)
        _expand(s0, q0, _B, a0, cm_ref=c0)

    h3 = jnp.mean(t5_ref[...].astype(_F32) * a5[...][None], axis=1) - c5[...]
    h2 = jnp.mean(ty2_ref[...].astype(_F32) * ay[...][None], axis=1) - cy[...]
    h1 = jnp.mean(tz_ref[...] * az[...][None], axis=1) - cz[...]
    h0 = jnp.mean(tw_ref[...] * a0[...][None], axis=1) - c0[...]
    h = jnp.concatenate([h0, h1, h2, h3], axis=1)
    out_ref[...] = jnp.dot(h, w_ref[...], preferred_element_type=_F32) + b_ref[...]


def _bspec(l, d):
    return pl.BlockSpec((_TB, l, d), lambda i: (i, 0, 0))


def _cspec(r, c):
    return pl.BlockSpec((r, c), lambda i: (0, 0))


def _vmem(shape):
    return pltpu.VMEM(shape, _F32)


_CP = pltpu.CompilerParams(dimension_semantics=("arbitrary",))


def kernel(nodes_0, nodes_1, nodes_2, nodes_3, counts_0, counts_1,
           W_proj, b_proj, W30, b30, W31, b31, W32, b32,
           W20, b20, W21, b21, W10, b10, W00, b00, W, b):
    f = _F32
    cnt0 = counts_0[0].astype(f)[:, None]
    cnt1 = counts_1[0].astype(f)[:, None]

    s1_out = pl.pallas_call(
        _stage1,
        grid=(_STEPS,),
        in_specs=[
            _bspec(_C0, _F), _bspec(_C1, _F), _bspec(_N1, _F), _bspec(1, _F),
            _cspec(_F, 128),
            _cspec(128, 256), _cspec(128, 256), _cspec(128, 256), _cspec(128, 256),
            _cspec(1, 128),
            _cspec(1, 256), _cspec(1, 256), _cspec(1, 256), _cspec(1, 256),
        ],
        out_specs=[
            _bspec(_C0, 256), _bspec(_C1, 256), _bspec(_N1, 256), _bspec(1, 256),
            _cspec(_C0, 1), _cspec(_C0, 1),
            _cspec(_C1, 4), _cspec(_C1, 4),
            _cspec(_N1, 16), _cspec(_N1, 16),
            _cspec(1, 256), _cspec(1, 256),
        ],
        out_shape=[
            jax.ShapeDtypeStruct((_B, _C0, 256), _BF),
            jax.ShapeDtypeStruct((_B, _C1, 256), _BF),
            jax.ShapeDtypeStruct((_B, _N1, 256), f),
            jax.ShapeDtypeStruct((_B, 1, 256), f),
            jax.ShapeDtypeStruct((_C0, 1), f), jax.ShapeDtypeStruct((_C0, 1), f),
            jax.ShapeDtypeStruct((_C1, 4), f), jax.ShapeDtypeStruct((_C1, 4), f),
            jax.ShapeDtypeStruct((_N1, 16), f), jax.ShapeDtypeStruct((_N1, 16), f),
            jax.ShapeDtypeStruct((1, 256), f), jax.ShapeDtypeStruct((1, 256), f),
        ],
        scratch_shapes=[_vmem((_F, 256)), _vmem((_F, 256)), _vmem((_F, 256)),
                        _vmem((_F, 256)), _vmem((1, 256)), _vmem((1, 256)),
                        _vmem((1, 256)), _vmem((1, 256))],
        compiler_params=_CP,
    )(nodes_0, nodes_1, nodes_2, nodes_3, W_proj, W30, W20, W10, W00,
      b_proj[None, :], b30[None, :], b20[None, :], b10[None, :], b00[None, :])
    t1, ty1, tz1, tw1, s3, q3, s2, q2, sz, qz, s0, q0 = s1_out

    t3r, s31, q31 = pl.pallas_call(
        _stage2,
        grid=(_STEPS,),
        in_specs=[_bspec(_C0, 256), _cspec(_C0, 1), _cspec(_C0, 1),
                  _cspec(_C0, 1), _cspec(256, 512), _cspec(1, 512)],
        out_specs=[_bspec(_C1, 512), _cspec(_C0, 2), _cspec(_C0, 2)],
        out_shape=[
            jax.ShapeDtypeStruct((_B, _C1, 512), _BF),
            jax.ShapeDtypeStruct((_C0, 2), f), jax.ShapeDtypeStruct((_C0, 2), f),
        ],
        scratch_shapes=[_vmem((_C0, 256)), _vmem((_C0, 256))],
        compiler_params=_CP,
    )(t1, s3, q3, cnt0, W31, b31[None, :])

    t5, s5, q5, ty2, sy, qy = pl.pallas_call(
        _stage3,
        grid=(_STEPS,),
        in_specs=[_bspec(_C1, 512), _cspec(_C0, 2), _cspec(_C0, 2),
                  _cspec(_C1, 1), _cspec(512, 1024), _cspec(1, 1024),
                  _bspec(_C1, 256), _cspec(_C1, 4), _cspec(_C1, 4),
                  _cspec(256, 512), _cspec(1, 512)],
        out_specs=[_bspec(_C1, 1024), _cspec(_C1, 16), _cspec(_C1, 16),
                   _bspec(_C1, 512), _cspec(_C1, 8), _cspec(_C1, 8)],
        out_shape=[
            jax.ShapeDtypeStruct((_B, _C1, 1024), _BF),
            jax.ShapeDtypeStruct((_C1, 16), f), jax.ShapeDtypeStruct((_C1, 16), f),
            jax.ShapeDtypeStruct((_B, _C1, 512), _BF),
            jax.ShapeDtypeStruct((_C1, 8), f), jax.ShapeDtypeStruct((_C1, 8), f),
        ],
        scratch_shapes=[_vmem((_C1, 512)), _vmem((_C1, 512)),
                        _vmem((_C1, 256)), _vmem((_C1, 256))],
        compiler_params=_CP,
    )(t3r, s31, q31, cnt1, W32, b32[None, :], ty1, s2, q2, W21, b21[None, :])

    out2d = pl.pallas_call(
        _stage4,
        grid=(_STEPS,),
        in_specs=[_bspec(_C1, 1024), _cspec(_C1, 16), _cspec(_C1, 16),
                  _bspec(_C1, 512), _cspec(_C1, 8), _cspec(_C1, 8),
                  _bspec(_N1, 256), _cspec(_N1, 16), _cspec(_N1, 16),
                  _bspec(1, 256), _cspec(1, 256), _cspec(1, 256),
                  _cspec(2048, 2048), _cspec(1, 2048)],
        out_specs=pl.BlockSpec((_TB, 2048), lambda i: (i, 0)),
        out_shape=jax.ShapeDtypeStruct((_B, 2048), f),
        scratch_shapes=[_vmem((_C1, 1024)), _vmem((1, 1024)),
                        _vmem((_C1, 512)), _vmem((1, 512)),
                        _vmem((_N1, 256)), _vmem((1, 256)),
                        _vmem((1, 256)), _vmem((1, 256))],
        compiler_params=_CP,
    )(t5, s5, q5, ty2, sy, qy, tz1, sz, qz, tw1, s0, q0, W, b[None, :])

    return out2d.reshape(_B, 2048, 1)
